# baseline (speedup 1.0000x reference)
import functools

import jax
import jax.numpy as jnp
import numpy as np
from jax import lax
from jax.experimental import pallas as pl
from jax.experimental.pallas import tpu as pltpu
from jax.experimental.pallas import tpu_sc as plsc

_N = 262144
_D = 256
_NC = 2
_NS = 16
_NW = _NC * _NS
_L = 16
_RPW = _N // _NW
_R = 64
_NB = _RPW // _R
_CPR = _D // _L

_RC = np.float32(1.5 * 2.0 ** 23)
_HALF = np.float32(0.5)
_THRESH = np.float32(_D / 4.0)

_mesh = plsc.VectorSubcoreMesh(core_axis_name="c", subcore_axis_name="s")


def _quantize_row(xv, yv, r):
    acc = [jnp.zeros((_L,), jnp.float32) for _ in range(2)]
    y1s = []
    y2s = []
    for j in range(_CPR):
        v = xv[r, pl.ds(j * _L, _L)]
        y1 = (v + _RC) - _RC
        r1 = v - y1
        acc[j % 2] = acc[j % 2] + jnp.abs(r1)
        d = jnp.where(r1 >= 0.0, _HALF, -_HALF)
        y1s.append(y1)
        y2s.append(y1 + d)
    cs = plsc.cumsum(acc[0] + acc[1])
    tot = lax.gather(
        cs,
        jnp.full((_L, 1), _L - 1, jnp.int32),
        lax.GatherDimensionNumbers(
            offset_dims=(), collapsed_slice_dims=(0,), start_index_map=(0,)
        ),
        slice_sizes=(1,),
        mode=lax.GatherScatterMode.PROMISE_IN_BOUNDS,
    )
    use2 = tot > _THRESH
    for j in range(_CPR):
        yv[r, pl.ds(j * _L, _L)] = jnp.where(use2, y2s[j], y1s[j])


@functools.partial(
    pl.kernel,
    out_type=jax.ShapeDtypeStruct((_N, _D), jnp.float32),
    mesh=_mesh,
    scratch_types=[
        pltpu.VMEM((_R, _D), jnp.float32),
        pltpu.VMEM((_R, _D), jnp.float32),
        pltpu.VMEM((_R, _D), jnp.float32),
        pltpu.VMEM((_R, _D), jnp.float32),
        pltpu.SemaphoreType.DMA,
        pltpu.SemaphoreType.DMA,
        pltpu.SemaphoreType.DMA,
        pltpu.SemaphoreType.DMA,
    ],
    compiler_params=pltpu.CompilerParams(needs_layout_passes=False),
)
def _dn_quant(x_hbm, out_hbm, xv0, xv1, yv0, yv1, si0, si1, so0, so1):
    wid = lax.axis_index("s") * _NC + lax.axis_index("c")
    base = wid * _RPW
    xv = (xv0, xv1)
    yv = (yv0, yv1)
    si = (si0, si1)
    so = (so0, so1)

    def start_in(i, k):
        pltpu.async_copy(x_hbm.at[pl.ds(base + i * _R, _R)], xv[k], si[k])

    def start_out(i, k):
        pltpu.async_copy(yv[k], out_hbm.at[pl.ds(base + i * _R, _R)], so[k])

    def wait_in(i, k):
        pltpu.make_async_copy(
            x_hbm.at[pl.ds(base + i * _R, _R)], xv[k], si[k]
        ).wait()

    def wait_out(i, k):
        pltpu.make_async_copy(
            yv[k], out_hbm.at[pl.ds(base + i * _R, _R)], so[k]
        ).wait()

    start_in(0, 0)

    def pair_body(t, carry):
        for k in (0, 1):
            i = 2 * t + k

            @pl.when(i + 1 < _NB)
            def _():
                start_in(i + 1, (k + 1) % 2)

            wait_in(i, k)

            @pl.when(i >= 2)
            def _():
                wait_out(i - 2, k)

            @plsc.parallel_loop(0, _R, unroll=4)
            def _(r):
                _quantize_row(xv[k], yv[k], r)
            start_out(i, k)
        return carry

    lax.fori_loop(0, _NB // 2, pair_body, 0)
    wait_out(_NB - 2, 0)
    wait_out(_NB - 1, 1)


def kernel(x):
    return _dn_quant(x)


# speedup vs baseline: 2.2741x; 2.2741x over previous
import functools

import jax
import jax.numpy as jnp
import numpy as np
from jax import lax
from jax.experimental import pallas as pl
from jax.experimental.pallas import tpu as pltpu
from jax.experimental.pallas import tpu_sc as plsc

_N = 262144
_D = 256
_NC = 2
_NS = 16
_NW = _NC * _NS
_L = 16
_RPW = _N // _NW
_R = 64
_NB = _RPW // _R
_CPR = _D // _L

_RC = np.float32(1.5 * 2.0 ** 23)
_HALF = np.float32(0.5)
_THRESH = np.float32(_D / 4.0)

_mesh = plsc.VectorSubcoreMesh(core_axis_name="c", subcore_axis_name="s")


def _quantize_row(xv, yv, r):
    acc = [jnp.zeros((_L,), jnp.float32) for _ in range(2)]
    y1s = []
    y2s = []
    for j in range(_CPR):
        v = xv[r, pl.ds(j * _L, _L)]
        y1 = (v + _RC) - _RC
        r1 = v - y1
        acc[j % 2] = acc[j % 2] + jnp.abs(r1)
        d = jnp.where(r1 >= 0.0, _HALF, -_HALF)
        y1s.append(y1)
        y2s.append(y1 + d)
    cs = plsc.cumsum(acc[0] + acc[1])
    tot = lax.gather(
        cs,
        jnp.full((_L, 1), _L - 1, jnp.int32),
        lax.GatherDimensionNumbers(
            offset_dims=(), collapsed_slice_dims=(0,), start_index_map=(0,)
        ),
        slice_sizes=(1,),
        mode=lax.GatherScatterMode.PROMISE_IN_BOUNDS,
    )
    use2 = tot > _THRESH
    for j in range(_CPR):
        yv[r, pl.ds(j * _L, _L)] = jnp.where(use2, y2s[j], y1s[j])


@functools.partial(
    pl.kernel,
    out_type=jax.ShapeDtypeStruct((_N, _D), jnp.float32),
    mesh=_mesh,
    scratch_types=[
        pltpu.VMEM((_R, _D), jnp.float32),
        pltpu.VMEM((_R, _D), jnp.float32),
        pltpu.VMEM((_R, _D), jnp.float32),
        pltpu.VMEM((_R, _D), jnp.float32),
        pltpu.SemaphoreType.DMA,
        pltpu.SemaphoreType.DMA,
        pltpu.SemaphoreType.DMA,
        pltpu.SemaphoreType.DMA,
    ],
    compiler_params=pltpu.CompilerParams(needs_layout_passes=False),
)
def _dn_quant(x_hbm, out_hbm, xv0, xv1, yv0, yv1, si0, si1, so0, so1):
    wid = lax.axis_index("s") * _NC + lax.axis_index("c")
    base = wid * _RPW
    xv = (xv0, xv1)
    yv = (yv0, yv1)
    si = (si0, si1)
    so = (so0, so1)

    def start_in(i, k):
        pltpu.async_copy(x_hbm.at[pl.ds(base + i * _R, _R)], xv[k], si[k])

    def start_out(i, k):
        pltpu.async_copy(yv[k], out_hbm.at[pl.ds(base + i * _R, _R)], so[k])

    def wait_in(i, k):
        pltpu.make_async_copy(
            x_hbm.at[pl.ds(base + i * _R, _R)], xv[k], si[k]
        ).wait()

    def wait_out(i, k):
        pltpu.make_async_copy(
            yv[k], out_hbm.at[pl.ds(base + i * _R, _R)], so[k]
        ).wait()

    start_in(0, 0)

    def pair_body(t, carry):
        for k in (0, 1):
            i = 2 * t + k

            @pl.when(i + 1 < _NB)
            def _():
                start_in(i + 1, (k + 1) % 2)

            wait_in(i, k)

            @pl.when(i >= 2)
            def _():
                wait_out(i - 2, k)

            @plsc.parallel_loop(0, _R)
            def _(r):
                _quantize_row(xv[k], yv[k], r)
            start_out(i, k)
        return carry

    lax.fori_loop(0, _NB // 2, pair_body, 0)
    wait_out(_NB - 2, 0)
    wait_out(_NB - 1, 1)


def kernel(x):
    return _dn_quant(x)
